# baseline (speedup 1.0000x reference)
import functools

import jax
import jax.numpy as jnp
from jax import lax
from jax.experimental import pallas as pl
from jax.experimental.pallas import tpu as pltpu
from jax.experimental.pallas import tpu_sc as plsc

_NC = 2
_NS = 16
_NW = _NC * _NS

_CHUNK = 128
_K = 10


def _make_gather(B, D):
    assert B % (_NW * _CHUNK) == 0
    chunks_per_w = B // (_NW * _CHUNK)
    assert chunks_per_w % _K == 0
    outer = chunks_per_w // _K
    rows_per_group = _K * _CHUNK

    mesh = plsc.VectorSubcoreMesh(core_axis_name="c", subcore_axis_name="s")

    @functools.partial(
        pl.kernel,
        mesh=mesh,
        compiler_params=pltpu.CompilerParams(use_tc_tiling_on_sc=False),
        out_type=jax.ShapeDtypeStruct((B, D), jnp.float32),
        scratch_types=[
            pltpu.VMEM((chunks_per_w * _CHUNK,), jnp.int32),
            pltpu.VMEM((rows_per_group, D), jnp.float32),
            pltpu.SemaphoreType.DMA,
        ],
    )
    def gather_kernel(table_hbm, idx_hbm, out_hbm, idx_v, rows_v, sem):
        wid = lax.axis_index("s") * _NC + lax.axis_index("c")
        per_w = chunks_per_w * _CHUNK
        pltpu.sync_copy(idx_hbm.at[pl.ds(wid * per_w, per_w)], idx_v)
        row_base = wid * per_w

        def group(g, carry):
            descs = []
            for j in range(_K):
                descs.append(pltpu.async_copy(
                    table_hbm.at[idx_v.at[pl.ds((g * _K + j) * _CHUNK,
                                                _CHUNK)]],
                    rows_v.at[pl.ds(j * _CHUNK, _CHUNK)],
                    sem,
                ))
            for d in descs:
                d.wait()
            pltpu.sync_copy(
                rows_v,
                out_hbm.at[pl.ds(row_base + g * rows_per_group,
                                 rows_per_group)],
            )
            return carry

        lax.fori_loop(0, outer, group, 0)

    return gather_kernel


def kernel(input, table):
    seq, batch = input.shape
    _, embed = table.shape
    idx = input.reshape(-1)
    out = _make_gather(input.size, embed)(table, idx)
    return out.reshape(-1, batch, embed)
